# Initial kernel scaffold; baseline (speedup 1.0000x reference)
import jax
import jax.numpy as jnp
from jax.experimental import pallas as pl


def kernel(tokens, segment_ids, pos_ids, token_table, pos_table, seg_table):
    raise NotImplementedError("write your pallas kernel here")


# speedup vs baseline: 7.6550x; 7.6550x over previous
import functools

import jax
import jax.numpy as jnp
from jax import lax
from jax.experimental import pallas as pl
from jax.experimental.pallas import tpu as pltpu
from jax.experimental.pallas import tpu_sc as plsc


def _fuse_tables(pos_table, seg_table):
    num_seg, dim = seg_table.shape
    max_len = pos_table.shape[0]

    def body(pos_ref, seg_ref, out_ref):
        out_ref[...] = seg_ref[...][:, None, :] + pos_ref[...][None, :, :]

    out = pl.pallas_call(
        body,
        out_shape=jax.ShapeDtypeStruct((num_seg, max_len, dim), jnp.float32),
    )(pos_table, seg_table)
    return out.reshape(num_seg * max_len, dim)


def _sc_lookup(n_rows, dim, max_len):
    info = plsc.get_sparse_core_info()
    nc, ns, lanes = info.num_cores, info.num_subcores, info.num_lanes
    nw = nc * ns
    per_w = n_rows // nw
    CHUNK = 128
    n_chunks = per_w // CHUNK
    mesh = plsc.VectorSubcoreMesh(core_axis_name="c", subcore_axis_name="s")

    @functools.partial(
        pl.kernel,
        mesh=mesh,
        out_type=jax.ShapeDtypeStruct((n_rows, dim), jnp.float32),
        scratch_types=[
            pltpu.VMEM((CHUNK,), jnp.int32),
            pltpu.VMEM((CHUNK,), jnp.int32),
            pltpu.VMEM((CHUNK,), jnp.int32),
            pltpu.VMEM((CHUNK,), jnp.int32),
            pltpu.VMEM((CHUNK, dim), jnp.float32),
            pltpu.VMEM((CHUNK, dim), jnp.float32),
            pltpu.SemaphoreType.DMA,
            pltpu.SemaphoreType.DMA,
        ],
    )
    def k(tok_hbm, pos_hbm, seg_hbm, toktab_hbm, fustab_hbm, out_hbm,
          tokidx, posidx, segidx, fidx, tokbuf, fusbuf, sem1, sem2):
        wid = lax.axis_index("s") * nc + lax.axis_index("c")
        base = wid * per_w

        def chunk(ci, _):
            cb = base + ci * CHUNK
            pltpu.sync_copy(tok_hbm.at[pl.ds(cb, CHUNK)], tokidx)
            pltpu.sync_copy(pos_hbm.at[pl.ds(cb, CHUNK)], posidx)
            pltpu.sync_copy(seg_hbm.at[pl.ds(cb, CHUNK)], segidx)
            for i in range(CHUNK // lanes):
                sl = pl.ds(i * lanes, lanes)
                fidx[sl] = segidx[sl] * max_len + posidx[sl]
            g1 = pltpu.async_copy(toktab_hbm.at[tokidx], tokbuf, sem1)
            g2 = pltpu.async_copy(fustab_hbm.at[fidx], fusbuf, sem2)
            g1.wait()
            g2.wait()

            def addrow(r, _):
                for j in range(dim // lanes):
                    sl = pl.ds(j * lanes, lanes)
                    plsc.addupdate(tokbuf.at[r, sl], fusbuf[r, sl])
                return _

            lax.fori_loop(0, CHUNK, addrow, 0, unroll=False)
            pltpu.sync_copy(tokbuf, out_hbm.at[pl.ds(cb, CHUNK)])
            return _

        lax.fori_loop(0, n_chunks, chunk, 0, unroll=False)

    return k


def kernel(tokens, segment_ids, pos_ids, token_table, pos_table, seg_table):
    b, l = tokens.shape
    vocab, dim = token_table.shape
    max_len = pos_table.shape[0]
    n_rows = b * l

    fused = _fuse_tables(pos_table, seg_table)
    tok_flat = tokens.reshape(n_rows).astype(jnp.int32)
    pos_flat = pos_ids.reshape(n_rows).astype(jnp.int32)
    seg_flat = segment_ids.reshape(n_rows).astype(jnp.int32)

    out = _sc_lookup(n_rows, dim, max_len)(
        tok_flat, pos_flat, seg_flat, token_table, fused)
    return out.reshape(b, l, dim)


# speedup vs baseline: 14.8625x; 1.9415x over previous
import functools

import jax
import jax.numpy as jnp
from jax import lax
from jax.experimental import pallas as pl
from jax.experimental.pallas import tpu as pltpu
from jax.experimental.pallas import tpu_sc as plsc


def _prep_tc(pos_table, seg_table, pos2d, seg2d):
    num_seg, dim = seg_table.shape
    max_len = pos_table.shape[0]
    n_r, n_c = pos2d.shape

    def body(pos_ref, seg_ref, p2_ref, s2_ref, fus_ref, fidx_ref):
        fus_ref[...] = seg_ref[...][:, None, :] + pos_ref[...][None, :, :]
        fidx_ref[...] = s2_ref[...] * max_len + p2_ref[...]

    fused, fidx = pl.pallas_call(
        body,
        out_shape=(
            jax.ShapeDtypeStruct((num_seg, max_len, dim), jnp.float32),
            jax.ShapeDtypeStruct((n_r, n_c), jnp.int32),
        ),
    )(pos_table, seg_table, pos2d, seg2d)
    return fused.reshape(num_seg * max_len, dim), fidx


def _sc_lookup(n_rows, dim, n_fused):
    info = plsc.get_sparse_core_info()
    nc, ns, lanes = info.num_cores, info.num_subcores, info.num_lanes
    nw = nc * ns
    CHUNK = 128
    rows_per_w = n_rows // (nw * CHUNK)
    HALF = rows_per_w // 2
    mesh = plsc.VectorSubcoreMesh(core_axis_name="c", subcore_axis_name="s")

    @functools.partial(
        pl.kernel,
        mesh=mesh,
        out_type=jax.ShapeDtypeStruct((n_rows, dim), jnp.float32),
        scratch_types=[
            pltpu.VMEM((HALF, CHUNK), jnp.int32),
            pltpu.VMEM((HALF, CHUNK), jnp.int32),
            pltpu.VMEM((CHUNK, dim), jnp.float32),
            pltpu.VMEM((CHUNK, dim), jnp.float32),
            pltpu.VMEM((CHUNK, dim), jnp.float32),
            pltpu.VMEM((CHUNK, dim), jnp.float32),
            pltpu.VMEM((CHUNK, dim), jnp.float32),
            pltpu.VMEM((CHUNK, dim), jnp.float32),
            pltpu.SemaphoreType.DMA,
            pltpu.SemaphoreType.DMA,
            pltpu.SemaphoreType.DMA,
            pltpu.SemaphoreType.DMA,
        ],
    )
    def k(tok_hbm, fidx_hbm, toktab_hbm, fustab_hbm, out_hbm,
          tokidx, fidxv,
          tokbuf0, tokbuf1, fusbuf0, fusbuf1, outbuf0, outbuf1,
          gsem0, gsem1, wsem0, wsem1):
        tokbuf = (tokbuf0, tokbuf1)
        fusbuf = (fusbuf0, fusbuf1)
        outbuf = (outbuf0, outbuf1)
        gsem = (gsem0, gsem1)
        wsem = (wsem0, wsem1)

        cid = lax.axis_index("c")
        sid = lax.axis_index("s")
        wid = sid * nc + cid
        rowbase = wid * rows_per_w

        def fire_gathers(b, cg):
            pltpu.async_copy(toktab_hbm.at[tokidx.at[cg]], tokbuf[b], gsem[b])
            pltpu.async_copy(fustab_hbm.at[fidxv.at[cg]], fusbuf[b], gsem[b])

        def wait_gathers(b):
            pltpu.make_async_copy(toktab_hbm.at[pl.ds(0, CHUNK)], tokbuf[b],
                                  gsem[b]).wait()
            pltpu.make_async_copy(toktab_hbm.at[pl.ds(0, CHUNK)], fusbuf[b],
                                  gsem[b]).wait()

        def wait_write(b):
            pltpu.make_async_copy(outbuf[b], out_hbm.at[pl.ds(0, CHUNK)],
                                  wsem[b]).wait()

        for h in range(2):
            hb = rowbase + h * HALF
            pltpu.sync_copy(tok_hbm.at[pl.ds(hb, HALF)], tokidx)
            pltpu.sync_copy(fidx_hbm.at[pl.ds(hb, HALF)], fidxv)
            fire_gathers(0, 0)
            fire_gathers(1, 1)

            def body(kk, carry):
                for b in range(2):
                    cg = 2 * kk + b
                    gidx = h * HALF + cg
                    wait_gathers(b)

                    @pl.when(gidx >= 2)
                    def _():
                        wait_write(b)

                    def addrow(r, acc):
                        for j in range(dim // lanes):
                            sl = pl.ds(j * lanes, lanes)
                            outbuf[b][r, sl] = tokbuf[b][r, sl] + fusbuf[b][r, sl]
                        return acc

                    lax.fori_loop(0, CHUNK, addrow, 0, unroll=False)

                    @pl.when(cg + 2 < HALF)
                    def _():
                        fire_gathers(b, cg + 2)

                    pltpu.async_copy(
                        outbuf[b],
                        out_hbm.at[pl.ds((rowbase + gidx) * CHUNK, CHUNK)],
                        wsem[b])
                return carry

            lax.fori_loop(0, HALF // 2, body, 0, unroll=False)

        wait_write(0)
        wait_write(1)

    return k


def kernel(tokens, segment_ids, pos_ids, token_table, pos_table, seg_table):
    b, l = tokens.shape
    vocab, dim = token_table.shape
    max_len = pos_table.shape[0]
    num_seg = seg_table.shape[0]
    n_rows = b * l
    n_c = 128
    n_r = n_rows // n_c

    tok2d = tokens.reshape(n_r, n_c).astype(jnp.int32)
    pos2d = pos_ids.reshape(n_r, n_c).astype(jnp.int32)
    seg2d = segment_ids.reshape(n_r, n_c).astype(jnp.int32)

    fused, fidx2d = _prep_tc(pos_table, seg_table, pos2d, seg2d)

    out = _sc_lookup(n_rows, dim, num_seg * max_len)(
        tok2d, fidx2d, token_table, fused)
    return out.reshape(b, l, dim)
